# Initial kernel scaffold; baseline (speedup 1.0000x reference)
import functools

import jax
import jax.numpy as jnp
from jax import lax
from jax.experimental import pallas as pl
from jax.experimental.pallas import tpu as pltpu
from jax.experimental.pallas import tpu_sc as plsc

N = 10000
N_PAD = 10016
E = 320000
CHUNK = 128
EDGE_ROWS = 2528
E_PAD = EDGE_ROWS * CHUNK
NC = 2
NS = 16
ROWS_PER_TILE = EDGE_ROWS // (NC * NS)
DUMMY = 10008
IN_CH = 128
HID = 32

_MESH = plsc.VectorSubcoreMesh(core_axis_name="c", subcore_axis_name="s")


@functools.partial(
    pl.kernel,
    out_type=jax.ShapeDtypeStruct((NC, N_PAD, 1), jnp.float32),
    mesh=_MESH,
    scratch_types=[
        pltpu.VMEM((ROWS_PER_TILE, CHUNK), jnp.int32),
        pltpu.VMEM((CHUNK, 1), jnp.float32),
        pltpu.VMEM_SHARED((N_PAD, 1), jnp.float32),
    ],
)
def _sc_degree(col_hbm, ones_hbm, zeros_hbm, out_hbm, col_v, ones_v, acc):
    c = lax.axis_index("c")
    s = lax.axis_index("s")
    t = c * NS + s
    pltpu.sync_copy(col_hbm.at[pl.ds(t * ROWS_PER_TILE, ROWS_PER_TILE)], col_v)
    pltpu.sync_copy(ones_hbm, ones_v)

    @pl.when(s == 0)
    def _():
        pltpu.sync_copy(zeros_hbm, acc)

    plsc.subcore_barrier()

    def body(j, carry):
        pltpu.sync_copy(ones_v, acc.at[col_v.at[j]], add=True)
        return carry

    lax.fori_loop(0, ROWS_PER_TILE, body, 0)
    plsc.subcore_barrier()

    @pl.when(s == 0)
    def _():
        pltpu.sync_copy(acc, out_hbm.at[c])


@functools.partial(
    pl.kernel,
    out_type=jax.ShapeDtypeStruct((NC, N_PAD, HID), jnp.float32),
    mesh=_MESH,
    scratch_types=[
        pltpu.VMEM((ROWS_PER_TILE, CHUNK), jnp.int32),
        pltpu.VMEM((ROWS_PER_TILE, CHUNK), jnp.int32),
        pltpu.VMEM((CHUNK, HID), jnp.float32),
        pltpu.VMEM_SHARED((N_PAD, HID), jnp.float32),
        pltpu.SemaphoreType.DMA,
    ],
)
def _sc_edge_aggregate(y_hbm, row_hbm, col_hbm, zeros_hbm, out_hbm,
                       row_v, col_v, gbuf, acc, sem):
    c = lax.axis_index("c")
    s = lax.axis_index("s")
    t = c * NS + s
    pltpu.sync_copy(row_hbm.at[pl.ds(t * ROWS_PER_TILE, ROWS_PER_TILE)], row_v)
    pltpu.sync_copy(col_hbm.at[pl.ds(t * ROWS_PER_TILE, ROWS_PER_TILE)], col_v)
    nrows = N_PAD // NS
    pltpu.sync_copy(zeros_hbm.at[pl.ds(s * nrows, nrows)],
                    acc.at[pl.ds(s * nrows, nrows)])
    plsc.subcore_barrier()

    def body(j, carry):
        pltpu.async_copy(y_hbm.at[row_v.at[j]], gbuf, sem).wait()
        pltpu.sync_copy(gbuf, acc.at[col_v.at[j]], add=True)
        return carry

    lax.fori_loop(0, ROWS_PER_TILE, body, 0)
    plsc.subcore_barrier()
    pltpu.sync_copy(acc.at[pl.ds(s * nrows, nrows)],
                    out_hbm.at[c, pl.ds(s * nrows, nrows)])


_RB = 2504


def _tc_prep(x, W1, d0, d1):
    def body(x_ref, w_ref, d0_ref, d1_ref, y_ref, dis_ref):
        dis = lax.rsqrt(d0_ref[...] + d1_ref[...] + 1.0)
        xw = jnp.dot(x_ref[...], w_ref[...], preferred_element_type=jnp.float32)
        y_ref[...] = xw * dis
        dis_ref[...] = dis

    return pl.pallas_call(
        body,
        grid=(N_PAD // _RB,),
        in_specs=[
            pl.BlockSpec((_RB, IN_CH), lambda i: (i, 0)),
            pl.BlockSpec((IN_CH, HID), lambda i: (0, 0)),
            pl.BlockSpec((_RB, 1), lambda i: (i, 0)),
            pl.BlockSpec((_RB, 1), lambda i: (i, 0)),
        ],
        out_specs=[
            pl.BlockSpec((_RB, HID), lambda i: (i, 0)),
            pl.BlockSpec((_RB, 1), lambda i: (i, 0)),
        ],
        out_shape=[
            jax.ShapeDtypeStruct((N_PAD, HID), jnp.float32),
            jax.ShapeDtypeStruct((N_PAD, 1), jnp.float32),
        ],
    )(x, W1, d0, d1)


def _tc_mid(s1, y1, dis, b1, W2):
    def body(s_ref, y_ref, dis_ref, b_ref, w_ref, y2_ref):
        dis = dis_ref[...]
        agg = (s_ref[0] + s_ref[1] + y_ref[...]) * dis + b_ref[...]
        h1 = jnp.maximum(agg, 0.0)
        y2_ref[...] = jnp.dot(h1, w_ref[...],
                              preferred_element_type=jnp.float32) * dis

    return pl.pallas_call(
        body,
        grid=(N_PAD // _RB,),
        in_specs=[
            pl.BlockSpec((NC, _RB, HID), lambda i: (0, i, 0)),
            pl.BlockSpec((_RB, HID), lambda i: (i, 0)),
            pl.BlockSpec((_RB, 1), lambda i: (i, 0)),
            pl.BlockSpec((1, HID), lambda i: (0, 0)),
            pl.BlockSpec((HID, HID), lambda i: (0, 0)),
        ],
        out_specs=pl.BlockSpec((_RB, HID), lambda i: (i, 0)),
        out_shape=jax.ShapeDtypeStruct((N_PAD, HID), jnp.float32),
    )(s1, y1, dis, b1, W2)


def _tc_final(s2, y2, dis, b2, Wl1, bl1, Wl2, bl2):
    def body(s_ref, y_ref, dis_ref, b_ref, wl1_ref, bl1_ref, wl2_ref,
             bl2_ref, o_ref):
        dis = dis_ref[...]
        h2 = jnp.maximum((s_ref[0] + s_ref[1] + y_ref[...]) * dis + b_ref[...],
                         0.0)
        h3 = jnp.maximum(
            jnp.dot(h2, wl1_ref[...], preferred_element_type=jnp.float32)
            + bl1_ref[...], 0.0)
        o_ref[...] = jnp.dot(h3, wl2_ref[...],
                             preferred_element_type=jnp.float32) + bl2_ref[...]

    return pl.pallas_call(
        body,
        grid=(N_PAD // _RB,),
        in_specs=[
            pl.BlockSpec((NC, _RB, HID), lambda i: (0, i, 0)),
            pl.BlockSpec((_RB, HID), lambda i: (i, 0)),
            pl.BlockSpec((_RB, 1), lambda i: (i, 0)),
            pl.BlockSpec((1, HID), lambda i: (0, 0)),
            pl.BlockSpec((HID, HID), lambda i: (0, 0)),
            pl.BlockSpec((1, HID), lambda i: (0, 0)),
            pl.BlockSpec((HID, 1), lambda i: (0, 0)),
            pl.BlockSpec((1, 1), lambda i: (0, 0)),
        ],
        out_specs=pl.BlockSpec((_RB, 1), lambda i: (i, 0)),
        out_shape=jax.ShapeDtypeStruct((N_PAD, 1), jnp.float32),
    )(s2, y2, dis, b2, Wl1, bl1, Wl2, bl2)


def kernel(x, edge_index, W1, b1, W2, b2, Wl1, bl1, Wl2, bl2):
    ei = edge_index.astype(jnp.int32)
    padv = jnp.full((E_PAD - E,), DUMMY, jnp.int32)
    row2d = jnp.concatenate([ei[0], padv]).reshape(EDGE_ROWS, CHUNK)
    col2d = jnp.concatenate([ei[1], padv]).reshape(EDGE_ROWS, CHUNK)
    x_pad = jnp.zeros((N_PAD, IN_CH), jnp.float32).at[:N].set(
        x.astype(jnp.float32))
    zeros_h = jnp.zeros((N_PAD, HID), jnp.float32)
    zeros_1 = jnp.zeros((N_PAD, 1), jnp.float32)
    ones_1 = jnp.ones((CHUNK, 1), jnp.float32)

    deg_p = _sc_degree(col2d, ones_1, zeros_1)
    y1, dis = _tc_prep(x_pad, W1, deg_p[0], deg_p[1])
    s1 = _sc_edge_aggregate(y1, row2d, col2d, zeros_h)
    y2 = _tc_mid(s1, y1, dis, b1.reshape(1, HID), W2)
    s2 = _sc_edge_aggregate(y2, row2d, col2d, zeros_h)
    out = _tc_final(s2, y2, dis, b2.reshape(1, HID), Wl1,
                    bl1.reshape(1, HID), Wl2, bl2.reshape(1, 1))
    return out[:N]


# speedup vs baseline: 21.2397x; 21.2397x over previous
import functools

import jax
import jax.numpy as jnp
from jax import lax
from jax.experimental import pallas as pl
from jax.experimental.pallas import tpu as pltpu
from jax.experimental.pallas import tpu_sc as plsc

N = 10000
N_PAD = 10112
E = 320000
CHUNK = 128
EDGE_ROWS = 2560
E_PAD = EDGE_ROWS * CHUNK
NC = 2
NS = 16
ROWS_PER_TILE = EDGE_ROWS // (NC * NS)
DUMMY = 10048
IN_CH = 128
HID = 32
DEG_W = 8


@functools.cache
def _sc_kernels():
    mesh = plsc.VectorSubcoreMesh(core_axis_name="c", subcore_axis_name="s",
                                  num_cores=NC, num_subcores=NS)

    sc_params = pltpu.CompilerParams(use_tc_tiling_on_sc=False)

    @functools.partial(
        pl.kernel,
        out_type=jax.ShapeDtypeStruct((NC, N_PAD, DEG_W), jnp.float32),
        mesh=mesh,
        compiler_params=sc_params,
        scratch_types=[
            pltpu.VMEM((ROWS_PER_TILE, CHUNK), jnp.int32),
            pltpu.VMEM((CHUNK, DEG_W), jnp.float32),
            pltpu.VMEM_SHARED((N_PAD, DEG_W), jnp.float32),
        ],
    )
    def sc_degree(col_hbm, ones_hbm, zeros_hbm, out_hbm, col_v, ones_v, acc):
        c = lax.axis_index("c")
        s = lax.axis_index("s")
        t = c * NS + s
        pltpu.sync_copy(col_hbm.at[pl.ds(t * ROWS_PER_TILE, ROWS_PER_TILE)],
                        col_v)
        pltpu.sync_copy(ones_hbm, ones_v)
        nrows = N_PAD // NS
        pltpu.sync_copy(zeros_hbm.at[pl.ds(s * nrows, nrows)],
                        acc.at[pl.ds(s * nrows, nrows)])
        plsc.subcore_barrier()

        def body(j, carry):
            pltpu.sync_copy(ones_v, acc.at[col_v.at[j]], add=True)
            return carry

        lax.fori_loop(0, ROWS_PER_TILE, body, 0)
        plsc.subcore_barrier()
        pltpu.sync_copy(acc.at[pl.ds(s * nrows, nrows)],
                        out_hbm.at[c, pl.ds(s * nrows, nrows)])

    @functools.partial(
        pl.kernel,
        out_type=jax.ShapeDtypeStruct((NC, N_PAD, HID), jnp.float32),
        mesh=mesh,
        compiler_params=sc_params,
        scratch_types=[
            pltpu.VMEM((ROWS_PER_TILE, CHUNK), jnp.int32),
            pltpu.VMEM((ROWS_PER_TILE, CHUNK), jnp.int32),
            pltpu.VMEM((CHUNK, HID), jnp.float32),
            pltpu.VMEM_SHARED((N_PAD, HID), jnp.float32),
            pltpu.SemaphoreType.DMA,
        ],
    )
    def sc_edge_aggregate(y_hbm, row_hbm, col_hbm, zeros_hbm, out_hbm,
                          row_v, col_v, gbuf, acc, sem):
        c = lax.axis_index("c")
        s = lax.axis_index("s")
        t = c * NS + s
        pltpu.sync_copy(row_hbm.at[pl.ds(t * ROWS_PER_TILE, ROWS_PER_TILE)],
                        row_v)
        pltpu.sync_copy(col_hbm.at[pl.ds(t * ROWS_PER_TILE, ROWS_PER_TILE)],
                        col_v)
        nrows = N_PAD // NS
        pltpu.sync_copy(zeros_hbm.at[pl.ds(s * nrows, nrows)],
                        acc.at[pl.ds(s * nrows, nrows)])
        plsc.subcore_barrier()

        def body(j, carry):
            pltpu.async_copy(y_hbm.at[row_v.at[j]], gbuf, sem).wait()
            pltpu.sync_copy(gbuf, acc.at[col_v.at[j]], add=True)
            return carry

        lax.fori_loop(0, ROWS_PER_TILE, body, 0)
        plsc.subcore_barrier()
        pltpu.sync_copy(acc.at[pl.ds(s * nrows, nrows)],
                        out_hbm.at[c, pl.ds(s * nrows, nrows)])

    return sc_degree, sc_edge_aggregate


_RB = 2528


def _tc_prep(x, W1, d0, d1):
    def body(x_ref, w_ref, d0_ref, d1_ref, y_ref, dis_ref):
        dis = lax.rsqrt(d0_ref[...] + d1_ref[...] + 1.0)
        xw = jnp.dot(x_ref[...], w_ref[...], preferred_element_type=jnp.float32)
        y_ref[...] = xw * dis
        dis_ref[...] = dis

    return pl.pallas_call(
        body,
        grid=(N_PAD // _RB,),
        in_specs=[
            pl.BlockSpec((_RB, IN_CH), lambda i: (i, 0)),
            pl.BlockSpec((IN_CH, HID), lambda i: (0, 0)),
            pl.BlockSpec((_RB, 1), lambda i: (i, 0)),
            pl.BlockSpec((_RB, 1), lambda i: (i, 0)),
        ],
        out_specs=[
            pl.BlockSpec((_RB, HID), lambda i: (i, 0)),
            pl.BlockSpec((_RB, 1), lambda i: (i, 0)),
        ],
        out_shape=[
            jax.ShapeDtypeStruct((N_PAD, HID), jnp.float32),
            jax.ShapeDtypeStruct((N_PAD, 1), jnp.float32),
        ],
    )(x, W1, d0, d1)


def _tc_mid(s1, y1, dis, b1, W2):
    def body(s_ref, y_ref, dis_ref, b_ref, w_ref, y2_ref):
        dis = dis_ref[...]
        agg = (s_ref[0] + s_ref[1] + y_ref[...]) * dis + b_ref[...]
        h1 = jnp.maximum(agg, 0.0)
        y2_ref[...] = jnp.dot(h1, w_ref[...],
                              preferred_element_type=jnp.float32) * dis

    return pl.pallas_call(
        body,
        grid=(N_PAD // _RB,),
        in_specs=[
            pl.BlockSpec((NC, _RB, HID), lambda i: (0, i, 0)),
            pl.BlockSpec((_RB, HID), lambda i: (i, 0)),
            pl.BlockSpec((_RB, 1), lambda i: (i, 0)),
            pl.BlockSpec((1, HID), lambda i: (0, 0)),
            pl.BlockSpec((HID, HID), lambda i: (0, 0)),
        ],
        out_specs=pl.BlockSpec((_RB, HID), lambda i: (i, 0)),
        out_shape=jax.ShapeDtypeStruct((N_PAD, HID), jnp.float32),
    )(s1, y1, dis, b1, W2)


def _tc_final(s2, y2, dis, b2, Wl1, bl1, Wl2, bl2):
    def body(s_ref, y_ref, dis_ref, b_ref, wl1_ref, bl1_ref, wl2_ref,
             bl2_ref, o_ref):
        dis = dis_ref[...]
        h2 = jnp.maximum((s_ref[0] + s_ref[1] + y_ref[...]) * dis + b_ref[...],
                         0.0)
        h3 = jnp.maximum(
            jnp.dot(h2, wl1_ref[...], preferred_element_type=jnp.float32)
            + bl1_ref[...], 0.0)
        o_ref[...] = jnp.dot(h3, wl2_ref[...],
                             preferred_element_type=jnp.float32) + bl2_ref[...]

    return pl.pallas_call(
        body,
        grid=(N_PAD // _RB,),
        in_specs=[
            pl.BlockSpec((NC, _RB, HID), lambda i: (0, i, 0)),
            pl.BlockSpec((_RB, HID), lambda i: (i, 0)),
            pl.BlockSpec((_RB, 1), lambda i: (i, 0)),
            pl.BlockSpec((1, HID), lambda i: (0, 0)),
            pl.BlockSpec((HID, HID), lambda i: (0, 0)),
            pl.BlockSpec((1, HID), lambda i: (0, 0)),
            pl.BlockSpec((HID, 1), lambda i: (0, 0)),
            pl.BlockSpec((1, 1), lambda i: (0, 0)),
        ],
        out_specs=pl.BlockSpec((_RB, 1), lambda i: (i, 0)),
        out_shape=jax.ShapeDtypeStruct((N_PAD, 1), jnp.float32),
    )(s2, y2, dis, b2, Wl1, bl1, Wl2, bl2)


def kernel(x, edge_index, W1, b1, W2, b2, Wl1, bl1, Wl2, bl2):
    ei = edge_index.astype(jnp.int32)
    padv = jnp.full((E_PAD - E,), DUMMY, jnp.int32)
    row2d = jnp.concatenate([ei[0], padv]).reshape(EDGE_ROWS, CHUNK)
    col2d = jnp.concatenate([ei[1], padv]).reshape(EDGE_ROWS, CHUNK)
    x_pad = jnp.zeros((N_PAD, IN_CH), jnp.float32).at[:N].set(
        x.astype(jnp.float32))
    zeros_h = jnp.zeros((N_PAD, HID), jnp.float32)
    zeros_d = jnp.zeros((N_PAD, DEG_W), jnp.float32)
    ones_d = jnp.ones((CHUNK, DEG_W), jnp.float32)

    sc_degree, sc_edge_aggregate = _sc_kernels()
    deg_p = sc_degree(col2d, ones_d, zeros_d)
    y1, dis = _tc_prep(x_pad, W1, deg_p[0, :, :1], deg_p[1, :, :1])
    s1 = sc_edge_aggregate(y1, row2d, col2d, zeros_h)
    y2 = _tc_mid(s1, y1, dis, b1.reshape(1, HID), W2)
    s2 = sc_edge_aggregate(y2, row2d, col2d, zeros_h)
    out = _tc_final(s2, y2, dis, b2.reshape(1, HID), Wl1,
                    bl1.reshape(1, HID), Wl2, bl2.reshape(1, 1))
    return out[:N]


# speedup vs baseline: 24.8631x; 1.1706x over previous
import functools

import jax
import jax.numpy as jnp
from jax import lax
from jax.experimental import pallas as pl
from jax.experimental.pallas import tpu as pltpu
from jax.experimental.pallas import tpu_sc as plsc

N = 10000
N_PAD = 10112
E = 320000
CHUNK = 128
EDGE_ROWS = 2560
E_PAD = EDGE_ROWS * CHUNK
NC = 2
NS = 16
ROWS_PER_TILE = EDGE_ROWS // (NC * NS)
DUMMY = 10048
IN_CH = 128
HID = 32
DEG_W = 8


@functools.cache
def _sc_kernels():
    mesh = plsc.VectorSubcoreMesh(core_axis_name="c", subcore_axis_name="s",
                                  num_cores=NC, num_subcores=NS)

    sc_params = pltpu.CompilerParams(use_tc_tiling_on_sc=False)

    @functools.partial(
        pl.kernel,
        out_type=jax.ShapeDtypeStruct((NC, N_PAD, DEG_W), jnp.float32),
        mesh=mesh,
        compiler_params=sc_params,
        scratch_types=[
            pltpu.VMEM((ROWS_PER_TILE, CHUNK), jnp.int32),
            pltpu.VMEM((CHUNK, DEG_W), jnp.float32),
            pltpu.VMEM_SHARED((N_PAD, DEG_W), jnp.float32),
        ],
    )
    def sc_degree(col_hbm, ones_hbm, zeros_hbm, out_hbm, col_v, ones_v, acc):
        c = lax.axis_index("c")
        s = lax.axis_index("s")
        t = c * NS + s
        pltpu.sync_copy(col_hbm.at[pl.ds(t * ROWS_PER_TILE, ROWS_PER_TILE)],
                        col_v)
        pltpu.sync_copy(ones_hbm, ones_v)
        nrows = N_PAD // NS
        pltpu.sync_copy(zeros_hbm.at[pl.ds(s * nrows, nrows)],
                        acc.at[pl.ds(s * nrows, nrows)])
        plsc.subcore_barrier()

        def body(j, carry):
            pltpu.sync_copy(ones_v, acc.at[col_v.at[j]], add=True)
            return carry

        lax.fori_loop(0, ROWS_PER_TILE, body, 0)
        plsc.subcore_barrier()
        pltpu.sync_copy(acc.at[pl.ds(s * nrows, nrows)],
                        out_hbm.at[c, pl.ds(s * nrows, nrows)])

    @functools.partial(
        pl.kernel,
        out_type=jax.ShapeDtypeStruct((NC, N_PAD, HID), jnp.float32),
        mesh=mesh,
        compiler_params=sc_params,
        scratch_types=[
            pltpu.VMEM((ROWS_PER_TILE, CHUNK), jnp.int32),
            pltpu.VMEM((ROWS_PER_TILE, CHUNK), jnp.int32),
            pltpu.VMEM((CHUNK, HID), jnp.float32),
            pltpu.VMEM((CHUNK, HID), jnp.float32),
            pltpu.VMEM_SHARED((N_PAD, HID), jnp.float32),
            pltpu.SemaphoreType.DMA,
            pltpu.SemaphoreType.DMA,
            pltpu.SemaphoreType.DMA,
            pltpu.SemaphoreType.DMA,
        ],
    )
    def sc_edge_aggregate(y_hbm, row_hbm, col_hbm, zeros_hbm, out_hbm,
                          row_v, col_v, gba, gbb, acc, gsa, gsb, ssa, ssb):
        c = lax.axis_index("c")
        s = lax.axis_index("s")
        t = c * NS + s
        nrows = N_PAD // NS
        esl = pl.ds(t * ROWS_PER_TILE, ROWS_PER_TILE)
        asl = pl.ds(s * nrows, nrows)
        pltpu.async_copy(row_hbm.at[esl], row_v, gsa)
        pltpu.async_copy(col_hbm.at[esl], col_v, gsb)
        pltpu.async_copy(zeros_hbm.at[asl], acc.at[asl], ssa)
        pltpu.make_async_copy(row_hbm.at[esl], row_v, gsa).wait()
        pltpu.make_async_copy(col_hbm.at[esl], col_v, gsb).wait()
        pltpu.make_async_copy(zeros_hbm.at[asl], acc.at[asl], ssa).wait()
        plsc.subcore_barrier()

        pltpu.async_copy(y_hbm.at[row_v.at[0]], gba, gsa)

        def body(j2, carry):
            b0 = j2 * 2
            pltpu.async_copy(y_hbm.at[row_v.at[b0 + 1]], gbb, gsb)
            pltpu.make_async_copy(y_hbm.at[row_v.at[b0]], gba, gsa).wait()
            pltpu.async_copy(gba, acc.at[col_v.at[b0]], ssa, add=True)
            pltpu.make_async_copy(y_hbm.at[row_v.at[b0 + 1]], gbb, gsb).wait()
            pltpu.async_copy(gbb, acc.at[col_v.at[b0 + 1]], ssb, add=True)
            pltpu.make_async_copy(gba, acc.at[col_v.at[b0]], ssa).wait()

            @pl.when(b0 + 2 < ROWS_PER_TILE)
            def _():
                pltpu.async_copy(y_hbm.at[row_v.at[b0 + 2]], gba, gsa)

            pltpu.make_async_copy(gbb, acc.at[col_v.at[b0 + 1]], ssb).wait()
            return carry

        lax.fori_loop(0, ROWS_PER_TILE // 2, body, 0)
        plsc.subcore_barrier()
        pltpu.sync_copy(acc.at[pl.ds(s * nrows, nrows)],
                        out_hbm.at[c, pl.ds(s * nrows, nrows)])

    return sc_degree, sc_edge_aggregate


_RB = 2528


def _tc_prep(x, W1, d0, d1):
    def body(x_ref, w_ref, d0_ref, d1_ref, y_ref, dis_ref):
        dis = lax.rsqrt(d0_ref[...] + d1_ref[...] + 1.0)
        xw = jnp.dot(x_ref[...], w_ref[...], preferred_element_type=jnp.float32)
        y_ref[...] = xw * dis
        dis_ref[...] = dis

    return pl.pallas_call(
        body,
        grid=(N_PAD // _RB,),
        in_specs=[
            pl.BlockSpec((_RB, IN_CH), lambda i: (i, 0)),
            pl.BlockSpec((IN_CH, HID), lambda i: (0, 0)),
            pl.BlockSpec((_RB, 1), lambda i: (i, 0)),
            pl.BlockSpec((_RB, 1), lambda i: (i, 0)),
        ],
        out_specs=[
            pl.BlockSpec((_RB, HID), lambda i: (i, 0)),
            pl.BlockSpec((_RB, 1), lambda i: (i, 0)),
        ],
        out_shape=[
            jax.ShapeDtypeStruct((N_PAD, HID), jnp.float32),
            jax.ShapeDtypeStruct((N_PAD, 1), jnp.float32),
        ],
    )(x, W1, d0, d1)


def _tc_mid(s1, y1, dis, b1, W2):
    def body(s_ref, y_ref, dis_ref, b_ref, w_ref, y2_ref):
        dis = dis_ref[...]
        agg = (s_ref[0] + s_ref[1] + y_ref[...]) * dis + b_ref[...]
        h1 = jnp.maximum(agg, 0.0)
        y2_ref[...] = jnp.dot(h1, w_ref[...],
                              preferred_element_type=jnp.float32) * dis

    return pl.pallas_call(
        body,
        grid=(N_PAD // _RB,),
        in_specs=[
            pl.BlockSpec((NC, _RB, HID), lambda i: (0, i, 0)),
            pl.BlockSpec((_RB, HID), lambda i: (i, 0)),
            pl.BlockSpec((_RB, 1), lambda i: (i, 0)),
            pl.BlockSpec((1, HID), lambda i: (0, 0)),
            pl.BlockSpec((HID, HID), lambda i: (0, 0)),
        ],
        out_specs=pl.BlockSpec((_RB, HID), lambda i: (i, 0)),
        out_shape=jax.ShapeDtypeStruct((N_PAD, HID), jnp.float32),
    )(s1, y1, dis, b1, W2)


def _tc_final(s2, y2, dis, b2, Wl1, bl1, Wl2, bl2):
    def body(s_ref, y_ref, dis_ref, b_ref, wl1_ref, bl1_ref, wl2_ref,
             bl2_ref, o_ref):
        dis = dis_ref[...]
        h2 = jnp.maximum((s_ref[0] + s_ref[1] + y_ref[...]) * dis + b_ref[...],
                         0.0)
        h3 = jnp.maximum(
            jnp.dot(h2, wl1_ref[...], preferred_element_type=jnp.float32)
            + bl1_ref[...], 0.0)
        o_ref[...] = jnp.dot(h3, wl2_ref[...],
                             preferred_element_type=jnp.float32) + bl2_ref[...]

    return pl.pallas_call(
        body,
        grid=(N_PAD // _RB,),
        in_specs=[
            pl.BlockSpec((NC, _RB, HID), lambda i: (0, i, 0)),
            pl.BlockSpec((_RB, HID), lambda i: (i, 0)),
            pl.BlockSpec((_RB, 1), lambda i: (i, 0)),
            pl.BlockSpec((1, HID), lambda i: (0, 0)),
            pl.BlockSpec((HID, HID), lambda i: (0, 0)),
            pl.BlockSpec((1, HID), lambda i: (0, 0)),
            pl.BlockSpec((HID, 1), lambda i: (0, 0)),
            pl.BlockSpec((1, 1), lambda i: (0, 0)),
        ],
        out_specs=pl.BlockSpec((_RB, 1), lambda i: (i, 0)),
        out_shape=jax.ShapeDtypeStruct((N_PAD, 1), jnp.float32),
    )(s2, y2, dis, b2, Wl1, bl1, Wl2, bl2)


def kernel(x, edge_index, W1, b1, W2, b2, Wl1, bl1, Wl2, bl2):
    ei = edge_index.astype(jnp.int32)
    padv = jnp.full((E_PAD - E,), DUMMY, jnp.int32)
    row2d = jnp.concatenate([ei[0], padv]).reshape(EDGE_ROWS, CHUNK)
    col2d = jnp.concatenate([ei[1], padv]).reshape(EDGE_ROWS, CHUNK)
    x_pad = jnp.zeros((N_PAD, IN_CH), jnp.float32).at[:N].set(
        x.astype(jnp.float32))
    zeros_h = jnp.zeros((N_PAD, HID), jnp.float32)
    zeros_d = jnp.zeros((N_PAD, DEG_W), jnp.float32)
    ones_d = jnp.ones((CHUNK, DEG_W), jnp.float32)

    sc_degree, sc_edge_aggregate = _sc_kernels()
    deg_p = sc_degree(col2d, ones_d, zeros_d)
    y1, dis = _tc_prep(x_pad, W1, deg_p[0, :, :1], deg_p[1, :, :1])
    s1 = sc_edge_aggregate(y1, row2d, col2d, zeros_h)
    y2 = _tc_mid(s1, y1, dis, b1.reshape(1, HID), W2)
    s2 = sc_edge_aggregate(y2, row2d, col2d, zeros_h)
    out = _tc_final(s2, y2, dis, b2.reshape(1, HID), Wl1,
                    bl1.reshape(1, HID), Wl2, bl2.reshape(1, 1))
    return out[:N]
